# baseline (speedup 1.0000x reference)
import jax
import jax.numpy as jnp
from jax.experimental import pallas as pl
from jax.experimental.pallas import tpu as pltpu

_S_BLK = 8


def _body(x_ref, states_ref, len_ref):
    states_ref[...] = x_ref[...]
    len_ref[...] = jnp.zeros_like(len_ref)


def kernel(batch):
    S, B, D = batch.shape
    x2 = batch.reshape(S, B * D)
    states, lengths = pl.pallas_call(
        _body,
        grid=(S // _S_BLK,),
        in_specs=[pl.BlockSpec((_S_BLK, B * D), lambda i: (i, 0))],
        out_specs=[
            pl.BlockSpec((_S_BLK, B * D), lambda i: (i, 0)),
            pl.BlockSpec((1, B), lambda i: (0, 0)),
        ],
        out_shape=[
            jax.ShapeDtypeStruct((S, B * D), batch.dtype),
            jax.ShapeDtypeStruct((1, B), jnp.int32),
        ],
        compiler_params=pltpu.CompilerParams(
            dimension_semantics=("parallel",),
        ),
    )(x2)
    return states.reshape(B, S, D), lengths.reshape(B)


# speedup vs baseline: 1.3248x; 1.3248x over previous
import jax
import jax.numpy as jnp
from jax.experimental import pallas as pl
from jax.experimental.pallas import tpu as pltpu

_B_BLK = 128


def _body(x_ref, states_ref, len_ref):
    x = x_ref[...]
    states_ref[...] = jnp.transpose(x, (1, 0, 2))
    rows = jnp.sum(x, axis=2)
    nz = jnp.where(rows != 0.0, 1.0, 0.0)
    len_ref[...] = jnp.sum(nz, axis=0)[None, :].astype(jnp.int32)


def kernel(batch):
    S, B, D = batch.shape
    states, lengths = pl.pallas_call(
        _body,
        grid=(B // _B_BLK,),
        in_specs=[pl.BlockSpec((S, _B_BLK, D), lambda i: (0, i, 0))],
        out_specs=[
            pl.BlockSpec((_B_BLK, S, D), lambda i: (i, 0, 0)),
            pl.BlockSpec((1, _B_BLK), lambda i: (0, i)),
        ],
        out_shape=[
            jax.ShapeDtypeStruct((B, S, D), batch.dtype),
            jax.ShapeDtypeStruct((1, B), jnp.int32),
        ],
        compiler_params=pltpu.CompilerParams(
            dimension_semantics=("parallel",),
        ),
    )(batch)
    return states, lengths.reshape(B)
